# Initial kernel scaffold; baseline (speedup 1.0000x reference)
import jax
import jax.numpy as jnp
from jax.experimental import pallas as pl


def kernel(text_a_ids, text_b_ids, emb_table, W1, b1, W2, b2):
    raise NotImplementedError("write your pallas kernel here")


# speedup vs baseline: 11.1350x; 11.1350x over previous
import functools

import jax
import jax.numpy as jnp
from jax import lax
from jax.experimental import pallas as pl
from jax.experimental.pallas import tpu as pltpu
from jax.experimental.pallas import tpu_sc as plsc

B = 4096
S = 200
E = 64
NC = 2
NS = 16
NW = NC * NS
BPW = B // NW

_SPLIT0 = 96
_SPLIT1 = S - _SPLIT0

_mesh = plsc.VectorSubcoreMesh(core_axis_name="c", subcore_axis_name="s")


@functools.partial(
    pl.kernel,
    out_type=jax.ShapeDtypeStruct((B, 2 * E), jnp.float32),
    mesh=_mesh,
    compiler_params=pltpu.CompilerParams(use_tc_tiling_on_sc=False),
    scratch_types=[
        pltpu.VMEM((BPW, S), jnp.int32),
        pltpu.VMEM((S, E), jnp.float32),
        pltpu.VMEM((BPW, 2 * E), jnp.float32),
        pltpu.SemaphoreType.DMA,
    ],
)
def _pool_kernel(a_hbm, b_hbm, table_hbm, out_hbm, idx_v, rows_v, out_v, sem):
    wid = lax.axis_index("s") * NC + lax.axis_index("c")
    base = wid * BPW

    def do_text(ids_hbm, col0):
        pltpu.sync_copy(ids_hbm.at[pl.ds(base, BPW)], idx_v)

        def seg_body(bi, carry):
            cp0 = pltpu.async_copy(
                table_hbm.at[idx_v.at[bi, pl.ds(0, _SPLIT0)]],
                rows_v.at[pl.ds(0, _SPLIT0)], sem)
            cp1 = pltpu.async_copy(
                table_hbm.at[idx_v.at[bi, pl.ds(_SPLIT0, _SPLIT1)]],
                rows_v.at[pl.ds(_SPLIT0, _SPLIT1)], sem)
            cp0.wait()
            cp1.wait()

            def acc_body(r, accs):
                return tuple(accs[i] + rows_v[r, pl.ds(16 * i, 16)]
                             for i in range(4))

            zero = jnp.zeros((16,), jnp.float32)
            accs = lax.fori_loop(0, S, acc_body, (zero, zero, zero, zero))
            for i in range(4):
                out_v[bi, pl.ds(col0 + 16 * i, 16)] = accs[i]
            return carry

        lax.fori_loop(0, BPW, seg_body, 0)

    do_text(a_hbm, 0)
    do_text(b_hbm, E)
    pltpu.sync_copy(out_v, out_hbm.at[pl.ds(base, BPW)])


def _mlp_body(x_ref, w1_ref, b1_ref, w2_ref, b2_ref, o_ref):
    h = jnp.dot(x_ref[...], w1_ref[...], preferred_element_type=jnp.float32)
    h = jnp.maximum(h + b1_ref[...], 0.0)
    z = jnp.dot(h, w2_ref[...], preferred_element_type=jnp.float32)
    o_ref[...] = jax.nn.sigmoid(z + b2_ref[...])


def _mlp(pooled, W1, b1, W2, b2):
    H = W1.shape[1]
    bm = 512
    return pl.pallas_call(
        _mlp_body,
        grid=(B // bm,),
        in_specs=[
            pl.BlockSpec((bm, 2 * E), lambda i: (i, 0)),
            pl.BlockSpec((2 * E, H), lambda i: (0, 0)),
            pl.BlockSpec((1, H), lambda i: (0, 0)),
            pl.BlockSpec((H, 1), lambda i: (0, 0)),
            pl.BlockSpec((1, 1), lambda i: (0, 0)),
        ],
        out_specs=pl.BlockSpec((bm, 1), lambda i: (i, 0)),
        out_shape=jax.ShapeDtypeStruct((B, 1), jnp.float32),
    )(pooled, W1, b1.reshape(1, H), W2, b2.reshape(1, 1))


def kernel(text_a_ids, text_b_ids, emb_table, W1, b1, W2, b2):
    pooled = _pool_kernel(text_a_ids, text_b_ids, emb_table)
    return _mlp(pooled, W1, b1, W2, b2)


# speedup vs baseline: 20.0346x; 1.7993x over previous
import functools

import jax
import jax.numpy as jnp
from jax import lax
from jax.experimental import pallas as pl
from jax.experimental.pallas import tpu as pltpu
from jax.experimental.pallas import tpu_sc as plsc

B = 4096
S = 200
E = 64
NC = 2
NS = 16
NW = NC * NS
BPW = B // NW

_SPLIT0 = 96
_SPLIT1 = S - _SPLIT0

_mesh = plsc.VectorSubcoreMesh(core_axis_name="c", subcore_axis_name="s")


@functools.partial(
    pl.kernel,
    out_type=jax.ShapeDtypeStruct((B, 2 * E), jnp.float32),
    mesh=_mesh,
    compiler_params=pltpu.CompilerParams(use_tc_tiling_on_sc=False),
    scratch_types=[
        pltpu.VMEM((2 * BPW, S), jnp.int32),
        pltpu.VMEM((2, S, E), jnp.float32),
        pltpu.VMEM((BPW, 2 * E), jnp.float32),
        pltpu.SemaphoreType.DMA,
        pltpu.SemaphoreType.DMA,
    ],
)
def _pool_kernel(a_hbm, b_hbm, table_hbm, out_hbm, idx_v, rows_v, out_v,
                 sem0, sem1):
    wid = lax.axis_index("s") * NC + lax.axis_index("c")
    base = wid * BPW
    sems = (sem0, sem1)

    def gather_seg(row, p, sem):
        pltpu.make_async_copy(
            table_hbm.at[idx_v.at[row, pl.ds(0, _SPLIT0)]],
            rows_v.at[p, pl.ds(0, _SPLIT0)], sem).start()
        pltpu.make_async_copy(
            table_hbm.at[idx_v.at[row, pl.ds(_SPLIT0, _SPLIT1)]],
            rows_v.at[p, pl.ds(_SPLIT0, _SPLIT1)], sem).start()

    def wait_seg(p, sem):
        pltpu.make_async_copy(
            table_hbm.at[pl.ds(0, S)], rows_v.at[p], sem).wait()

    def acc_seg(p, g, col0):
        def body(r, accs):
            return tuple(accs[i] + rows_v[p, r, pl.ds(16 * i, 16)]
                         for i in range(4))
        z = jnp.zeros((16,), jnp.float32)
        accs = lax.fori_loop(0, S, body, (z, z, z, z), unroll=8)
        for i in range(4):
            out_v[g, pl.ds(col0 + 16 * i, 16)] = accs[i]

    pltpu.sync_copy(a_hbm.at[pl.ds(base, BPW)], idx_v.at[pl.ds(0, BPW)])
    pltpu.sync_copy(b_hbm.at[pl.ds(base, BPW)], idx_v.at[pl.ds(BPW, BPW)])

    gather_seg(0, 0, sems[0])

    def g_body(g, carry):
        gather_seg(BPW + g, 1, sems[1])
        wait_seg(0, sems[0])
        acc_seg(0, g, 0)

        @pl.when(g + 1 < BPW)
        def _():
            gather_seg(g + 1, 0, sems[0])

        wait_seg(1, sems[1])
        acc_seg(1, g, E)
        return carry

    lax.fori_loop(0, BPW, g_body, 0)
    pltpu.sync_copy(out_v, out_hbm.at[pl.ds(base, BPW)])


def _mlp_body(x_ref, w1_ref, b1_ref, w2_ref, b2_ref, o_ref):
    h = jnp.dot(x_ref[...], w1_ref[...], preferred_element_type=jnp.float32)
    h = jnp.maximum(h + b1_ref[...], 0.0)
    z = jnp.dot(h, w2_ref[...], preferred_element_type=jnp.float32)
    o_ref[...] = jax.nn.sigmoid(z + b2_ref[...])


def _mlp(pooled, W1, b1, W2, b2):
    H = W1.shape[1]
    bm = 512
    return pl.pallas_call(
        _mlp_body,
        grid=(B // bm,),
        in_specs=[
            pl.BlockSpec((bm, 2 * E), lambda i: (i, 0)),
            pl.BlockSpec((2 * E, H), lambda i: (0, 0)),
            pl.BlockSpec((1, H), lambda i: (0, 0)),
            pl.BlockSpec((H, 1), lambda i: (0, 0)),
            pl.BlockSpec((1, 1), lambda i: (0, 0)),
        ],
        out_specs=pl.BlockSpec((bm, 1), lambda i: (i, 0)),
        out_shape=jax.ShapeDtypeStruct((B, 1), jnp.float32),
    )(pooled, W1, b1.reshape(1, H), W2, b2.reshape(1, 1))


def kernel(text_a_ids, text_b_ids, emb_table, W1, b1, W2, b2):
    pooled = _pool_kernel(text_a_ids, text_b_ids, emb_table)
    return _mlp(pooled, W1, b1, W2, b2)


# speedup vs baseline: 21.6450x; 1.0804x over previous
import functools

import jax
import jax.numpy as jnp
import numpy as np
from jax import lax
from jax.experimental import pallas as pl
from jax.experimental.pallas import tpu as pltpu
from jax.experimental.pallas import tpu_sc as plsc

B = 4096
S = 200
E = 64
NC = 2
NS = 16
NW = NC * NS
BPW = B // NW

_SPLIT0 = 96
_SPLIT1 = S - _SPLIT0

_mesh = plsc.VectorSubcoreMesh(core_axis_name="c", subcore_axis_name="s")


@functools.partial(
    pl.kernel,
    out_type=jax.ShapeDtypeStruct((B, 2 * E), jnp.float32),
    mesh=_mesh,
    compiler_params=pltpu.CompilerParams(
        use_tc_tiling_on_sc=False, needs_layout_passes=False),
    scratch_types=[
        pltpu.VMEM((2 * BPW, S), jnp.int32),
        pltpu.VMEM((2, S, E), jnp.bfloat16),
        pltpu.VMEM((BPW, 2 * E), jnp.float32),
        pltpu.SemaphoreType.DMA,
        pltpu.SemaphoreType.DMA,
    ],
)
def _pool_kernel(a_hbm, b_hbm, table_hbm, out_hbm, idx_v, rows_v, out_v,
                 sem0, sem1):
    wid = lax.axis_index("s") * NC + lax.axis_index("c")
    base = wid * BPW
    sems = (sem0, sem1)

    def gather_seg(row, p, sem):
        pltpu.make_async_copy(
            table_hbm.at[idx_v.at[row, pl.ds(0, _SPLIT0)]],
            rows_v.at[p, pl.ds(0, _SPLIT0)], sem).start()
        pltpu.make_async_copy(
            table_hbm.at[idx_v.at[row, pl.ds(_SPLIT0, _SPLIT1)]],
            rows_v.at[p, pl.ds(_SPLIT0, _SPLIT1)], sem).start()

    def wait_seg(p, sem):
        pltpu.make_async_copy(
            table_hbm.at[pl.ds(0, S)], rows_v.at[p], sem).wait()

    def acc_seg(p, g, col0):
        mask_hi = jnp.uint32(0xFFFF0000)

        def body(r, accs):
            new = []
            for j in range(2):
                w = rows_v[p, r, pl.ds(32 * j, 32)]
                u = plsc.bitcast(w, jnp.uint32)
                ev = plsc.bitcast(u << 16, jnp.float32)
                od = plsc.bitcast(u & mask_hi, jnp.float32)
                new.append(accs[2 * j] + ev)
                new.append(accs[2 * j + 1] + od)
            return tuple(new)

        z = jnp.zeros((16,), jnp.float32)
        accs = lax.fori_loop(0, S, body, (z, z, z, z), unroll=8)
        for i in range(4):
            out_v[g, pl.ds(col0 + 16 * i, 16)] = accs[i]

    pltpu.sync_copy(a_hbm.at[pl.ds(base, BPW)], idx_v.at[pl.ds(0, BPW)])
    pltpu.sync_copy(b_hbm.at[pl.ds(base, BPW)], idx_v.at[pl.ds(BPW, BPW)])

    gather_seg(0, 0, sems[0])

    def g_body(g, carry):
        gather_seg(BPW + g, 1, sems[1])
        wait_seg(0, sems[0])
        acc_seg(0, g, 0)

        @pl.when(g + 1 < BPW)
        def _():
            gather_seg(g + 1, 0, sems[0])

        wait_seg(1, sems[1])
        acc_seg(1, g, E)
        return carry

    lax.fori_loop(0, BPW, g_body, 0)
    pltpu.sync_copy(out_v, out_hbm.at[pl.ds(base, BPW)])


def _mlp_body(x_ref, w1_ref, b1_ref, w2_ref, b2_ref, o_ref):
    h = jnp.dot(x_ref[...], w1_ref[...], preferred_element_type=jnp.float32)
    h = jnp.maximum(h + b1_ref[...], 0.0)
    z = jnp.dot(h, w2_ref[...], preferred_element_type=jnp.float32)
    o_ref[...] = jax.nn.sigmoid(z + b2_ref[...])


def _mlp(pooled, W1, b1, W2, b2):
    H = W1.shape[1]
    bm = 512
    return pl.pallas_call(
        _mlp_body,
        grid=(B // bm,),
        in_specs=[
            pl.BlockSpec((bm, 2 * E), lambda i: (i, 0)),
            pl.BlockSpec((2 * E, H), lambda i: (0, 0)),
            pl.BlockSpec((1, H), lambda i: (0, 0)),
            pl.BlockSpec((H, 1), lambda i: (0, 0)),
            pl.BlockSpec((1, 1), lambda i: (0, 0)),
        ],
        out_specs=pl.BlockSpec((bm, 1), lambda i: (i, 0)),
        out_shape=jax.ShapeDtypeStruct((B, 1), jnp.float32),
    )(pooled, W1, b1.reshape(1, H), W2, b2.reshape(1, 1))


def _staged_perm():
    p = np.empty((2 * E,), dtype=np.int32)
    for c in range(2 * E):
        half, rem = divmod(c, E)
        j, within = divmod(rem, 32)
        h, k = divmod(within, 16)
        p[c] = half * E + 32 * j + 2 * k + h
    return p


_PERM = _staged_perm()


def kernel(text_a_ids, text_b_ids, emb_table, W1, b1, W2, b2):
    tbl16 = emb_table.astype(jnp.bfloat16)
    pooled = _pool_kernel(text_a_ids, text_b_ids, tbl16)
    W1p = jnp.take(W1, _PERM, axis=0)
    return _mlp(pooled, W1p, b1, W2, b2)


# speedup vs baseline: 28.2308x; 1.3043x over previous
import functools

import jax
import jax.numpy as jnp
import numpy as np
from jax import lax
from jax.experimental import pallas as pl
from jax.experimental.pallas import tpu as pltpu
from jax.experimental.pallas import tpu_sc as plsc

B = 4096
S = 200
E = 64
NC = 2
NS = 16
NW = NC * NS
BPW = B // NW

_SPLIT0 = 96
_SPLIT1 = S - _SPLIT0

_mesh = plsc.VectorSubcoreMesh(core_axis_name="c", subcore_axis_name="s")


@functools.partial(
    pl.kernel,
    out_type=jax.ShapeDtypeStruct((B, 2 * E), jnp.float32),
    mesh=_mesh,
    compiler_params=pltpu.CompilerParams(
        use_tc_tiling_on_sc=False, needs_layout_passes=False),
    scratch_types=[
        pltpu.VMEM((2 * BPW * S,), jnp.int32),
        pltpu.VMEM((4, S, E), jnp.bfloat16),
        pltpu.VMEM((BPW, 2 * E), jnp.float32),
        pltpu.SemaphoreType.DMA,
        pltpu.SemaphoreType.DMA,
        pltpu.SemaphoreType.DMA,
        pltpu.SemaphoreType.DMA,
    ],
)
def _pool_kernel(a_hbm, b_hbm, table_hbm, out_hbm, idx_v, rows_v, out_v,
                 sem0, sem1, sem2, sem3):
    wid = lax.axis_index("s") * NC + lax.axis_index("c")
    base = wid * BPW
    sems = (sem0, sem1, sem2, sem3)
    nseg = 2 * BPW

    def gather_seg(s, p, sem):
        off = s * S
        pltpu.make_async_copy(
            table_hbm.at[idx_v.at[pl.ds(off, _SPLIT0)]],
            rows_v.at[p, pl.ds(0, _SPLIT0)], sem).start()
        pltpu.make_async_copy(
            table_hbm.at[idx_v.at[pl.ds(off + _SPLIT0, _SPLIT1)]],
            rows_v.at[p, pl.ds(_SPLIT0, _SPLIT1)], sem).start()

    def wait_seg(p, sem):
        pltpu.make_async_copy(
            table_hbm.at[pl.ds(0, S)], rows_v.at[p], sem).wait()

    def acc_seg(p, s):
        t = s // BPW
        row = s - t * BPW
        col0 = t * E
        mask_hi = jnp.uint32(0xFFFF0000)

        def body(r, accs):
            new = []
            for j in range(2):
                w = rows_v[p, r, pl.ds(32 * j, 32)]
                u = plsc.bitcast(w, jnp.uint32)
                ev = plsc.bitcast(u << 16, jnp.float32)
                od = plsc.bitcast(u & mask_hi, jnp.float32)
                new.append(accs[2 * j] + ev)
                new.append(accs[2 * j + 1] + od)
            return tuple(new)

        z = jnp.zeros((16,), jnp.float32)
        accs = lax.fori_loop(0, S, body, (z, z, z, z), unroll=8)
        for i in range(4):
            out_v[row, pl.ds(col0 + 16 * i, 16)] = accs[i]

    pltpu.sync_copy(a_hbm.at[pl.ds(base * S, BPW * S)],
                    idx_v.at[pl.ds(0, BPW * S)])
    pltpu.sync_copy(b_hbm.at[pl.ds(base * S, BPW * S)],
                    idx_v.at[pl.ds(BPW * S, BPW * S)])

    for p in range(3):
        gather_seg(p, p, sems[p])

    def g_body(g4, carry):
        for q in range(4):
            s = 4 * g4 + q
            wait_seg(q, sems[q])
            pn = (q + 3) % 4

            @pl.when(s + 3 < nseg)
            def _():
                gather_seg(s + 3, pn, sems[pn])

            acc_seg(q, s)
        return carry

    lax.fori_loop(0, nseg // 4, g_body, 0)
    pltpu.sync_copy(out_v, out_hbm.at[pl.ds(base, BPW)])


def _mlp_body(x_ref, w1_ref, b1_ref, w2_ref, b2_ref, o_ref):
    h = jnp.dot(x_ref[...], w1_ref[...], preferred_element_type=jnp.float32)
    h = jnp.maximum(h + b1_ref[...], 0.0)
    z = jnp.dot(h, w2_ref[...], preferred_element_type=jnp.float32)
    o_ref[...] = jax.nn.sigmoid(z + b2_ref[...])


def _mlp(pooled, W1, b1, W2, b2):
    H = W1.shape[1]
    bm = 512
    return pl.pallas_call(
        _mlp_body,
        grid=(B // bm,),
        in_specs=[
            pl.BlockSpec((bm, 2 * E), lambda i: (i, 0)),
            pl.BlockSpec((2 * E, H), lambda i: (0, 0)),
            pl.BlockSpec((1, H), lambda i: (0, 0)),
            pl.BlockSpec((H, 1), lambda i: (0, 0)),
            pl.BlockSpec((1, 1), lambda i: (0, 0)),
        ],
        out_specs=pl.BlockSpec((bm, 1), lambda i: (i, 0)),
        out_shape=jax.ShapeDtypeStruct((B, 1), jnp.float32),
    )(pooled, W1, b1.reshape(1, H), W2, b2.reshape(1, 1))


def _staged_perm():
    p = np.empty((2 * E,), dtype=np.int32)
    for c in range(2 * E):
        half, rem = divmod(c, E)
        j, within = divmod(rem, 32)
        h, k = divmod(within, 16)
        p[c] = half * E + 32 * j + 2 * k + h
    return p


_PERM = _staged_perm()


def kernel(text_a_ids, text_b_ids, emb_table, W1, b1, W2, b2):
    tbl16 = emb_table.astype(jnp.bfloat16)
    pooled = _pool_kernel(text_a_ids.reshape(-1), text_b_ids.reshape(-1),
                          tbl16)
    W1p = jnp.take(W1, _PERM, axis=0)
    return _mlp(pooled, W1p, b1, W2, b2)
